# Initial kernel scaffold; baseline (speedup 1.0000x reference)
import jax
import jax.numpy as jnp
from jax.experimental import pallas as pl


def kernel(x, edge_index, edge_weight, W0, W1):
    raise NotImplementedError("write your pallas kernel here")


# speedup vs baseline: 6.2180x; 6.2180x over previous
import functools

import jax
import jax.numpy as jnp
from jax import lax
from jax.experimental import pallas as pl
from jax.experimental.pallas import tpu as pltpu
from jax.experimental.pallas import tpu_sc as plsc

N_NODES = 10000
N_EDGES = 320000
D = 128

NC = 2
NS = 16
NW = NC * NS
EDGES_PER_WORKER = N_EDGES // NW
K = 80
NCHUNK = EDGES_PER_WORKER // K
SUP = 25
NSUP = NCHUNK // SUP
N_PAD = 10240
ROWS_PER_TILE = N_PAD // NS
CB = ROWS_PER_TILE // K


def _mm_body(x_ref, w_ref, o_ref):
    o_ref[...] = jnp.dot(x_ref[...], w_ref[...],
                         preferred_element_type=jnp.float32)


def _matmul(x, w):
    return pl.pallas_call(
        _mm_body,
        out_shape=jax.ShapeDtypeStruct((x.shape[0], w.shape[1]), jnp.float32),
    )(x, w)


def _add_relu_mm_body(p_ref, w_ref, o_ref):
    h = jnp.maximum(p_ref[0] + p_ref[1], 0.0)
    o_ref[...] = jnp.dot(h, w_ref[...], preferred_element_type=jnp.float32)


def _add_relu_matmul(p, w):
    return pl.pallas_call(
        _add_relu_mm_body,
        out_shape=jax.ShapeDtypeStruct((p.shape[1], w.shape[1]), jnp.float32),
    )(p, w)


def _add_relu_body(p_ref, o_ref):
    o_ref[...] = jnp.maximum(p_ref[0] + p_ref[1], 0.0)


def _add_relu(p):
    return pl.pallas_call(
        _add_relu_body,
        out_shape=jax.ShapeDtypeStruct(p.shape[1:], jnp.float32),
    )(p)


def _spmm_body(h_hbm, src_hbm, dst_hbm, w_hbm, out_hbm,
               src_v, dst_v, w_v, rows_v, acc_sh, sem0):
    c = lax.axis_index("c")
    s = lax.axis_index("s")
    wid = s * NC + c

    zeros = jnp.zeros((16,), jnp.float32)

    def zero_row(i, carry):
        for d in range(D // 16):
            rows_v[i, pl.ds(d * 16, 16)] = zeros
        return carry

    lax.fori_loop(0, K, zero_row, 0)
    for j in range(CB):
        base = s * ROWS_PER_TILE + j * K
        pltpu.sync_copy(rows_v, acc_sh.at[pl.ds(base, K)])
    plsc.subcore_barrier()

    def sup_body(u, carry):
        pltpu.sync_copy(src_hbm.at[wid, u], src_v)
        pltpu.sync_copy(dst_hbm.at[wid, u], dst_v)
        pltpu.sync_copy(w_hbm.at[wid, u], w_v)

        def chunk_body(g, carry1):
            pltpu.async_copy(h_hbm.at[src_v.at[g]], rows_v, sem0).wait()

            def group_body(t, carry2):
                wv = w_v[g, pl.ds(t * 16, 16)]
                for j in range(16):
                    w = wv[j]
                    e = t * 16 + j
                    for d in range(D // 16):
                        sl = pl.ds(d * 16, 16)
                        rows_v[e, sl] = rows_v[e, sl] * w
                return carry2

            lax.fori_loop(0, K // 16, group_body, 0)
            pltpu.sync_copy(rows_v, acc_sh.at[dst_v.at[g]], add=True)
            return carry1

        lax.fori_loop(0, SUP, chunk_body, 0)
        return carry

    lax.fori_loop(0, NSUP, sup_body, 0)
    plsc.subcore_barrier()

    for j in range(CB):
        base = s * ROWS_PER_TILE + j * K
        pltpu.sync_copy(acc_sh.at[pl.ds(base, K)], rows_v)
        pltpu.sync_copy(rows_v, out_hbm.at[c, pl.ds(base, K)])


_spmm_call = functools.partial(
    pl.kernel,
    out_type=jax.ShapeDtypeStruct((NC, N_PAD, D), jnp.float32),
    mesh=plsc.VectorSubcoreMesh(core_axis_name="c", subcore_axis_name="s"),
    scratch_types=[
        pltpu.VMEM((SUP, K), jnp.int32),
        pltpu.VMEM((SUP, K), jnp.int32),
        pltpu.VMEM((SUP, K), jnp.float32),
        pltpu.VMEM((K, D), jnp.float32),
        pltpu.VMEM_SHARED((N_PAD, D), jnp.float32),
        pltpu.SemaphoreType.DMA,
    ],
)(_spmm_body)


def kernel(x, edge_index, edge_weight, W0, W1):
    src = edge_index[0].reshape(NW, NSUP, SUP, K)
    dst = edge_index[1].reshape(NW, NSUP, SUP, K)
    w = edge_weight.reshape(NW, NSUP, SUP, K)

    xw0 = _matmul(x, W0)
    p0 = _spmm_call(xw0, src, dst, w)
    hw1 = _add_relu_matmul(p0[:, :N_NODES], W1)
    p1 = _spmm_call(hw1, src, dst, w)
    return _add_relu(p1[:, :N_NODES])
